# Initial kernel scaffold; baseline (speedup 1.0000x reference)
import jax
import jax.numpy as jnp
from jax.experimental import pallas as pl


def kernel(time_x, length_q, embeddings_table):
    raise NotImplementedError("write your pallas kernel here")


# speedup vs baseline: 5.8155x; 5.8155x over previous
import functools
import jax
import jax.numpy as jnp
from jax import lax
from jax.experimental import pallas as pl
from jax.experimental.pallas import tpu as pltpu
from jax.experimental.pallas import tpu_sc as plsc

MAXR = 128


def _make_sc_kernel(L, D, rows):
    EXT = 2 * L
    NS = 16
    ext_per_tile = EXT // NS
    CH = 128
    n_ch = ext_per_tile // CH
    rows_per_worker = L // (2 * NS)

    mesh = plsc.VectorSubcoreMesh(core_axis_name="c", subcore_axis_name="s")

    @functools.partial(
        pl.kernel,
        mesh=mesh,
        out_type=jax.ShapeDtypeStruct((L, L, D), jnp.float32),
        compiler_params=pltpu.CompilerParams(use_tc_tiling_on_sc=False),
        scratch_types=[
            pltpu.VMEM_SHARED((EXT, D), jnp.float32),
            pltpu.VMEM((CH,), jnp.int32),
            pltpu.VMEM((CH, D), jnp.float32),
            pltpu.SemaphoreType.DMA,
        ],
    )
    def k(table_hbm, out_hbm, ext, idx, gbuf, sem):
        cid = lax.axis_index("c")
        sid = lax.axis_index("s")
        wid = sid * 2 + cid

        for c in range(n_ch):
            base = sid * ext_per_tile + c * CH
            for q in range(CH // 16):
                ii = lax.iota(jnp.int32, 16) + (base + q * 16 - (L - 1))
                idx[pl.ds(q * 16, 16)] = (
                    jnp.clip(ii, -MAXR, MAXR) + MAXR
                )
            pltpu.async_copy(table_hbm.at[idx], gbuf, sem).wait()
            pltpu.sync_copy(gbuf, ext.at[pl.ds(base, CH)])

        plsc.subcore_barrier()

        first = wid * rows_per_worker

        def body(t, carry):
            i = first + t
            pltpu.sync_copy(ext.at[pl.ds((L - 1) - i, L)], out_hbm.at[i])
            return carry

        lax.fori_loop(0, rows_per_worker, body, 0)

    return k


def kernel(time_x, length_q, embeddings_table):
    B, L, D = time_x.shape
    rows = embeddings_table.shape[0]
    out = _make_sc_kernel(L, D, rows)(embeddings_table)
    return jnp.broadcast_to(out[None], (B, L, L, D))
